# baseline (speedup 1.0000x reference)
import functools

import jax
import jax.numpy as jnp
from jax import lax
from jax.experimental import pallas as pl
from jax.experimental.pallas import tpu as pltpu
from jax.experimental.pallas import tpu_sc as plsc

_SCALE = 10.0
_D = 32
_L = 16
_NC = 2
_NS = 16
_NW = _NC * _NS
_CH = 128
_DT = _D // 8
_NBUF = 2


def _iota16():
    return jax.lax.iota(jnp.int32, _L)


@functools.cache
def _build_table_transform(nv: int, tail: int):
    full_cols = (nv - tail) // _CH
    base_cols = full_cols // _NW
    extra = full_cols - base_cols * _NW
    assert base_cols >= _NBUF

    mesh = plsc.VectorSubcoreMesh(core_axis_name="c", subcore_axis_name="s")

    @functools.partial(
        pl.kernel,
        out_type=jax.ShapeDtypeStruct((nv * _D,), jnp.float32),
        mesh=mesh,
        compiler_params=pltpu.CompilerParams(needs_layout_passes=False),
        scratch_types=[
            pltpu.VMEM((_NBUF, _DT, 8, _CH), jnp.float32),
            pltpu.VMEM((_NBUF, _CH * _D), jnp.float32),
            pltpu.VMEM((max(tail, 1) * _D,), jnp.float32),
            pltpu.SemaphoreType.DMA,
            pltpu.SemaphoreType.DMA,
            pltpu.SemaphoreType.DMA,
            pltpu.SemaphoreType.DMA,
        ],
    )
    def table_transform(wt_hbm, tail_hbm, out_hbm, in_v, out_v, tail_v,
                        g0, g1, s0, s1):
        gsem = (g0, g1)
        ssem = (s0, s1)
        wid = lax.axis_index("s") * _NC + lax.axis_index("c")
        ncols = base_cols + jnp.where(wid < extra, 1, 0).astype(jnp.int32)
        c0 = wid * base_cols + jnp.minimum(wid, extra)

        dvec = [_iota16() + p * _L for p in range(_D // _L)]

        def in_start(c, b):
            for dt in range(_DT):
                pltpu.async_copy(
                    wt_hbm.at[pl.ds(dt * 8, 8), pl.ds(c * _CH, _CH)],
                    in_v.at[b, dt],
                    gsem[b],
                )

        def in_wait(c, b):
            for dt in range(_DT):
                pltpu.make_async_copy(
                    wt_hbm.at[pl.ds(dt * 8, 8), pl.ds(c * _CH, _CH)],
                    in_v.at[b, dt],
                    gsem[b],
                ).wait()

        def out_start(c, b):
            pltpu.async_copy(
                out_v.at[b], out_hbm.at[pl.ds(c * (_CH * _D), _CH * _D)],
                ssem[b],
            )

        def out_wait(c, b):
            pltpu.make_async_copy(
                out_v.at[b], out_hbm.at[pl.ds(c * (_CH * _D), _CH * _D)],
                ssem[b],
            ).wait()

        def transpose_block(b):
            src = in_v.at[b]
            for j in range(_CH // 4):
                for v in range(2 * _DT):
                    u, p = divmod(v, 2)
                    g = plsc.load_gather(
                        src,
                        [
                            dvec[p] // 8,
                            lax.rem(dvec[p], 8),
                            jnp.full((_L,), 4 * j + u, jnp.int32),
                        ],
                    )
                    out_v[b, pl.ds(j * _CH + v * _L, _L)] = g * _SCALE

        in_start(c0, 0)
        in_start(c0 + 1, 1)

        def step(i, carry):
            for b in range(_NBUF):
                c = c0 + i * _NBUF + b
                in_wait(c, b)

                @pl.when(i >= 1)
                def _():
                    out_wait(c - _NBUF, b)

                transpose_block(b)
                out_start(c, b)

                @pl.when(c + _NBUF < c0 + ncols)
                def _():
                    in_start(c + _NBUF, b)

            return carry

        nsteps = ncols // _NBUF
        lax.fori_loop(0, nsteps, step, 0)

        @pl.when(nsteps * _NBUF < ncols)
        def _():
            c = c0 + nsteps * _NBUF
            in_wait(c, 0)
            out_wait(c - _NBUF, 0)
            transpose_block(0)
            out_start(c, 0)
            out_wait(c, 0)
            out_wait(c - _NBUF + 1, 1)

        @pl.when(nsteps * _NBUF == ncols)
        def _():
            out_wait(c0 + ncols - 2, 0)
            out_wait(c0 + ncols - 1, 1)

        if tail:
            @pl.when(wid == _NW - 1)
            def _():
                pltpu.sync_copy(tail_hbm, tail_v)
                tvec = [dvec[p] * tail for p in range(_D // _L)]
                for j in range(tail // 4):
                    for v in range(2 * _DT):
                        u, p = divmod(v, 2)
                        g = plsc.load_gather(
                            tail_v, [tvec[p] + (4 * j + u)]
                        )
                        out_v[0, pl.ds(j * _CH + v * _L, _L)] = g * _SCALE
                pltpu.sync_copy(
                    out_v.at[0, pl.ds(0, tail * _D)],
                    out_hbm.at[pl.ds(full_cols * _CH * _D, tail * _D)],
                )

    return table_transform


@functools.cache
def _build_gather(nb: int, na: int, nv: int):
    nchunks = nb * (na // _CH)
    assert nchunks % _NW == 0
    cpw = nchunks // _NW
    g_steps = cpw // _NBUF
    ta_n = na // _CH

    mesh = plsc.VectorSubcoreMesh(core_axis_name="c", subcore_axis_name="s")

    @functools.partial(
        pl.kernel,
        out_type=jax.ShapeDtypeStruct((nb, _DT, ta_n, 8, _CH), jnp.float32),
        mesh=mesh,
        compiler_params=pltpu.CompilerParams(
            needs_layout_passes=False, use_tc_tiling_on_sc=False
        ),
        scratch_types=[
            pltpu.VMEM((cpw, _CH), jnp.int32),
            pltpu.VMEM((_NBUF, _CH, _D), jnp.float32),
            pltpu.VMEM((_NBUF, _DT, 8, _CH), jnp.float32),
            pltpu.SemaphoreType.DMA,
            pltpu.SemaphoreType.DMA,
            pltpu.SemaphoreType.DMA,
            pltpu.SemaphoreType.DMA,
        ],
    )
    def scaled_gather(idx_hbm, tbl_hbm, out_hbm, idx_v, rows_v, blk_v,
                      g0, g1, s0, s1):
        gsem = (g0, g1)
        ssem = (s0, s1)
        wid = lax.axis_index("s") * _NC + lax.axis_index("c")
        cbase = wid * cpw

        pltpu.sync_copy(idx_hbm.at[pl.ds(cbase, cpw)], idx_v)

        row_ids = [_iota16() + k * _L for k in range(_CH // _L)]

        def gather_start(ci_local, b):
            pltpu.async_copy(
                tbl_hbm.at[idx_v.at[ci_local]], rows_v.at[b], gsem[b]
            )

        def gather_wait(ci_local, b):
            pltpu.make_async_copy(
                tbl_hbm.at[idx_v.at[ci_local]], rows_v.at[b], gsem[b]
            ).wait()

        def transpose_chunk(b):
            rows = rows_v.at[b]
            for dt in range(_DT):
                for di in range(8):
                    d = dt * 8 + di
                    for k in range(_CH // _L):
                        v = plsc.load_gather(
                            rows,
                            [row_ids[k], jnp.full((_L,), d, jnp.int32)],
                        )
                        blk_v[b, dt, di, pl.ds(k * _L, _L)] = v

        def store_start(ci_local, b):
            ci = cbase + ci_local
            bb = ci // ta_n
            ta = lax.rem(ci, ta_n)
            for dt in range(_DT):
                pltpu.async_copy(
                    blk_v.at[b, dt], out_hbm.at[bb, dt, ta], ssem[b]
                )

        def store_wait(ci_local, b):
            ci = cbase + ci_local
            bb = ci // ta_n
            ta = lax.rem(ci, ta_n)
            for dt in range(_DT):
                pltpu.make_async_copy(
                    blk_v.at[b, dt], out_hbm.at[bb, dt, ta], ssem[b]
                ).wait()

        for b in range(_NBUF):
            gather_start(b, b)

        def step(g, carry):
            for b in range(_NBUF):
                ci = g * _NBUF + b
                gather_wait(ci, b)

                @pl.when(g >= 1)
                def _():
                    store_wait(ci - _NBUF, b)

                transpose_chunk(b)
                store_start(ci, b)

                @pl.when(g < g_steps - 1)
                def _():
                    gather_start(ci + _NBUF, b)

            return carry

        lax.fori_loop(0, g_steps, step, 0)

        for b in range(_NBUF):
            store_wait((g_steps - 1) * _NBUF + b, b)

    return scaled_gather


def kernel(x, weight):
    na, nb = x.shape
    nv = weight.shape[0]
    tail = nv % _CH
    idx2d = x.T.reshape(nb * (na // _CH), _CH).astype(jnp.int32)
    tail_flat = weight[nv - tail:].T.reshape(tail * _D)
    w_flat = _build_table_transform(nv, tail)(weight.T, tail_flat)
    o5 = _build_gather(nb, na, nv)(idx2d, w_flat.reshape(nv, _D))
    out = jnp.transpose(o5, (2, 4, 0, 1, 3)).reshape(na, nb, _D)
    return out


# speedup vs baseline: 2.1903x; 2.1903x over previous
import functools

import jax
import jax.numpy as jnp
from jax import lax
from jax.experimental import pallas as pl
from jax.experimental.pallas import tpu as pltpu
from jax.experimental.pallas import tpu_sc as plsc

_SCALE = 10.0
_D = 32
_L = 16
_NC = 2
_NS = 16
_NW = _NC * _NS
_CH = 128
_DT = _D // 8
_NBUF = 2


def _iota16():
    return jax.lax.iota(jnp.int32, _L)


@functools.cache
def _build_table_transform(nv: int, tail: int):
    full_cols = (nv - tail) // _CH
    base_cols = full_cols // _NW
    extra = full_cols - base_cols * _NW
    assert base_cols >= _NBUF

    mesh = plsc.VectorSubcoreMesh(core_axis_name="c", subcore_axis_name="s")

    @functools.partial(
        pl.kernel,
        out_type=jax.ShapeDtypeStruct((nv * _D,), jnp.float32),
        mesh=mesh,
        compiler_params=pltpu.CompilerParams(needs_layout_passes=False),
        scratch_types=[
            pltpu.VMEM((_NBUF, _DT, 8, _CH), jnp.float32),
            pltpu.VMEM((_NBUF, _CH * _D), jnp.float32),
            pltpu.VMEM((max(tail, 1) * _D,), jnp.float32),
            pltpu.SemaphoreType.DMA,
            pltpu.SemaphoreType.DMA,
            pltpu.SemaphoreType.DMA,
            pltpu.SemaphoreType.DMA,
        ],
    )
    def table_transform(wt_hbm, tail_hbm, out_hbm, in_v, out_v, tail_v,
                        g0, g1, s0, s1):
        gsem = (g0, g1)
        ssem = (s0, s1)
        wid = lax.axis_index("s") * _NC + lax.axis_index("c")
        ncols = base_cols + jnp.where(wid < extra, 1, 0).astype(jnp.int32)
        c0 = wid * base_cols + jnp.minimum(wid, extra)

        dvec = [_iota16() + p * _L for p in range(_D // _L)]

        def in_start(c, b):
            for dt in range(_DT):
                pltpu.async_copy(
                    wt_hbm.at[pl.ds(dt * 8, 8), pl.ds(c * _CH, _CH)],
                    in_v.at[b, dt],
                    gsem[b],
                )

        def in_wait(c, b):
            for dt in range(_DT):
                pltpu.make_async_copy(
                    wt_hbm.at[pl.ds(dt * 8, 8), pl.ds(c * _CH, _CH)],
                    in_v.at[b, dt],
                    gsem[b],
                ).wait()

        def out_start(c, b):
            pltpu.async_copy(
                out_v.at[b], out_hbm.at[pl.ds(c * (_CH * _D), _CH * _D)],
                ssem[b],
            )

        def out_wait(c, b):
            pltpu.make_async_copy(
                out_v.at[b], out_hbm.at[pl.ds(c * (_CH * _D), _CH * _D)],
                ssem[b],
            ).wait()

        ih = _iota16() // 8
        il = lax.rem(_iota16(), 8)
        dtv = [ih + 2 * p for p in range(2)]

        def transpose_block(b):
            src = in_v.at[b]

            @plsc.parallel_loop(0, _CH // 4, unroll=4)
            def _(j):
                for u in range(4):
                    xs = jnp.broadcast_to(4 * j + u, (_L,)).astype(jnp.int32)
                    for p in range(2):
                        g = plsc.load_gather(src, [dtv[p], il, xs])
                        out_v[b, pl.ds(j * _CH + (2 * u + p) * _L, _L)] = (
                            g * _SCALE
                        )

        in_start(c0, 0)
        in_start(c0 + 1, 1)

        def step(i, carry):
            for b in range(_NBUF):
                c = c0 + i * _NBUF + b
                in_wait(c, b)

                @pl.when(i >= 1)
                def _():
                    out_wait(c - _NBUF, b)

                transpose_block(b)
                out_start(c, b)

                @pl.when(c + _NBUF < c0 + ncols)
                def _():
                    in_start(c + _NBUF, b)

            return carry

        nsteps = ncols // _NBUF
        lax.fori_loop(0, nsteps, step, 0)

        @pl.when(nsteps * _NBUF < ncols)
        def _():
            c = c0 + nsteps * _NBUF
            in_wait(c, 0)
            out_wait(c - _NBUF, 0)
            transpose_block(0)
            out_start(c, 0)
            out_wait(c, 0)
            out_wait(c - _NBUF + 1, 1)

        @pl.when(nsteps * _NBUF == ncols)
        def _():
            out_wait(c0 + ncols - 2, 0)
            out_wait(c0 + ncols - 1, 1)

        if tail:
            @pl.when(wid == _NW - 1)
            def _():
                pltpu.sync_copy(tail_hbm, tail_v)
                tvec = [dvec[p] * tail for p in range(_D // _L)]
                for j in range(tail // 4):
                    for v in range(2 * _DT):
                        u, p = divmod(v, 2)
                        g = plsc.load_gather(
                            tail_v, [tvec[p] + (4 * j + u)]
                        )
                        out_v[0, pl.ds(j * _CH + v * _L, _L)] = g * _SCALE
                pltpu.sync_copy(
                    out_v.at[0, pl.ds(0, tail * _D)],
                    out_hbm.at[pl.ds(full_cols * _CH * _D, tail * _D)],
                )

    return table_transform


@functools.cache
def _build_gather(nb: int, na: int, nv: int):
    nchunks = nb * (na // _CH)
    assert nchunks % _NW == 0
    cpw = nchunks // _NW
    g_steps = cpw // _NBUF
    ta_n = na // _CH

    mesh = plsc.VectorSubcoreMesh(core_axis_name="c", subcore_axis_name="s")

    @functools.partial(
        pl.kernel,
        out_type=jax.ShapeDtypeStruct((nb, _DT, ta_n, 8 * _CH), jnp.float32),
        mesh=mesh,
        compiler_params=pltpu.CompilerParams(
            needs_layout_passes=False, use_tc_tiling_on_sc=False
        ),
        scratch_types=[
            pltpu.VMEM((cpw, _CH), jnp.int32),
            pltpu.VMEM((_NBUF, _CH, _D), jnp.float32),
            pltpu.VMEM((_NBUF, _CH * _D), jnp.float32),
            pltpu.SemaphoreType.DMA,
            pltpu.SemaphoreType.DMA,
            pltpu.SemaphoreType.DMA,
            pltpu.SemaphoreType.DMA,
        ],
    )
    def scaled_gather(idx_hbm, tbl_hbm, out_hbm, idx_v, rows_v, blk_v,
                      g0, g1, s0, s1):
        gsem = (g0, g1)
        ssem = (s0, s1)
        wid = lax.axis_index("s") * _NC + lax.axis_index("c")
        cbase = wid * cpw

        pltpu.sync_copy(idx_hbm.at[pl.ds(cbase, cpw)], idx_v)

        iota = _iota16()

        def gather_start(ci_local, b):
            pltpu.async_copy(
                tbl_hbm.at[idx_v.at[ci_local]], rows_v.at[b], gsem[b]
            )

        def gather_wait(ci_local, b):
            pltpu.make_async_copy(
                tbl_hbm.at[idx_v.at[ci_local]], rows_v.at[b], gsem[b]
            ).wait()

        rowm = [iota + k * _L for k in range(_CH // _L)]

        def transpose_chunk(b):
            rows = rows_v.at[b]

            @plsc.parallel_loop(0, _D, unroll=4)
            def _(d):
                ds_ = jnp.broadcast_to(d, (_L,)).astype(jnp.int32)
                for k in range(_CH // _L):
                    v = plsc.load_gather(rows, [rowm[k], ds_])
                    blk_v[b, pl.ds(d * _CH + k * _L, _L)] = v

        def store_start(ci_local, b):
            ci = cbase + ci_local
            bb = ci // ta_n
            ta = lax.rem(ci, ta_n)
            for dt in range(_DT):
                pltpu.async_copy(
                    blk_v.at[b, pl.ds(dt * 8 * _CH, 8 * _CH)],
                    out_hbm.at[bb, dt, ta],
                    ssem[b],
                )

        def store_wait(ci_local, b):
            ci = cbase + ci_local
            bb = ci // ta_n
            ta = lax.rem(ci, ta_n)
            for dt in range(_DT):
                pltpu.make_async_copy(
                    blk_v.at[b, pl.ds(dt * 8 * _CH, 8 * _CH)],
                    out_hbm.at[bb, dt, ta],
                    ssem[b],
                ).wait()

        for b in range(_NBUF):
            gather_start(b, b)

        def step(g, carry):
            for b in range(_NBUF):
                ci = g * _NBUF + b
                gather_wait(ci, b)

                @pl.when(g >= 1)
                def _():
                    store_wait(ci - _NBUF, b)

                transpose_chunk(b)
                store_start(ci, b)

                @pl.when(g < g_steps - 1)
                def _():
                    gather_start(ci + _NBUF, b)

            return carry

        lax.fori_loop(0, g_steps, step, 0)

        for b in range(_NBUF):
            store_wait((g_steps - 1) * _NBUF + b, b)

    return scaled_gather


def kernel(x, weight):
    na, nb = x.shape
    nv = weight.shape[0]
    tail = nv % _CH
    idx2d = x.T.reshape(nb * (na // _CH), _CH).astype(jnp.int32)
    tail_flat = weight[nv - tail:].T.reshape(tail * _D)
    w_flat = _build_table_transform(nv, tail)(weight.T, tail_flat)
    o4 = _build_gather(nb, na, nv)(idx2d, w_flat.reshape(nv, _D))
    o5 = o4.reshape(nb, _DT, na // _CH, 8, _CH)
    out = jnp.transpose(o5, (2, 4, 0, 1, 3)).reshape(na, nb, _D)
    return out
